# baseline (speedup 1.0000x reference)
import functools

import jax
import jax.numpy as jnp
import numpy as np
from jax import lax
from jax.experimental import pallas as pl
from jax.experimental.pallas import tpu as pltpu
from jax.experimental.pallas import tpu_sc as plsc

_FIELD_DIMS = [100000] * 26
_OFFSETS = np.concatenate([[0], np.cumsum(_FIELD_DIMS)[:-1]]).astype(np.int32)

_F = 26
_D = 16
_B = 4096
_NW = 32
_BPW = _B // _NW


def _fm_body(idx_hbm, emb_hbm, fc_hbm, out_hbm, idx_v, rows_v, fc_v, out_v, sem):
    cid = lax.axis_index("c")
    sid = lax.axis_index("s")
    wid = sid * 2 + cid

    pltpu.sync_copy(idx_hbm.at[wid], idx_v)

    copies = []
    for f in range(_F):
        copies.append(pltpu.async_copy(emb_hbm.at[idx_v.at[f]], rows_v.at[f], sem))
    for f in range(_F):
        copies.append(pltpu.async_copy(fc_hbm.at[idx_v.at[f]], fc_v.at[f], sem))
    for cp in copies:
        cp.wait()

    lane = lax.iota(jnp.int32, _D)
    gdn = lax.GatherDimensionNumbers(
        offset_dims=(), collapsed_slice_dims=(0,), start_index_map=(0,))

    def lane_sum(v):
        for sh in (8, 4, 2, 1):
            perm = jnp.bitwise_xor(lane, sh)[:, None]
            v = v + lax.gather(
                v, perm, gdn, slice_sizes=(1,), unique_indices=True,
                indices_are_sorted=False,
                mode=lax.GatherScatterMode.PROMISE_IN_BOUNDS)
        return v

    for cb in range(_BPW // _D):
        lin = jnp.zeros((_D,), jnp.float32)
        for f in range(_F):
            lin = lin + fc_v[f, pl.ds(cb * _D, _D)]

        def chunk_body(k, rvec, cb=cb):
            i = cb * _D + k
            sacc = jnp.zeros((_D,), jnp.float32)
            qacc = jnp.zeros((_D,), jnp.float32)
            for f in range(_F):
                v = rows_v[f, i, :]
                sacc = sacc + v
                qacc = qacc + v * v
            r = lane_sum(sacc * sacc - qacc)
            return rvec + jnp.where(lane == k, r, 0.0)

        rvec = lax.fori_loop(0, _D, chunk_body, jnp.zeros((_D,), jnp.float32))
        out_v[pl.ds(cb * _D, _D)] = 0.5 * rvec + lin

    pltpu.sync_copy(out_v, out_hbm.at[pl.ds(wid * _BPW, _BPW)])


@jax.jit
def _fm_run(idxw, emb_table, fc_flat):
    mesh = plsc.VectorSubcoreMesh(core_axis_name="c", subcore_axis_name="s")
    return pl.kernel(
        _fm_body,
        out_type=jax.ShapeDtypeStruct((_B,), jnp.float32),
        mesh=mesh,
        scratch_types=[
            pltpu.VMEM((_F, _BPW), jnp.int32),
            pltpu.VMEM((_F, _BPW, _D), jnp.float32),
            pltpu.VMEM((_F, _BPW), jnp.float32),
            pltpu.VMEM((_BPW,), jnp.float32),
            pltpu.SemaphoreType.DMA,
        ],
        compiler_params=pltpu.CompilerParams(use_tc_tiling_on_sc=False),
    )(idxw, emb_table, fc_flat)


def kernel(x, emb_table, fc_table, bias):
    idx = x + jnp.asarray(_OFFSETS)[None, :]
    idxw = idx.reshape(_NW, _BPW, _F).transpose(0, 2, 1)
    y = _fm_run(idxw, emb_table, fc_table[:, 0])
    return y + bias[0]
